# Initial kernel scaffold; baseline (speedup 1.0000x reference)
import jax
import jax.numpy as jnp
from jax.experimental import pallas as pl


def kernel(x, masked_output, W, b, Wv, bv):
    raise NotImplementedError("write your pallas kernel here")


# speedup vs baseline: 4.7118x; 4.7118x over previous
import functools

import jax
import jax.numpy as jnp
from jax.experimental import pallas as pl

LATENT = 2048
VOCAB = 2048
ROWS = 512


def _ce_body(nrows_total, xs_ref, mask_ref, tgt_ref, w_ref, b_ref, out_ref):
    i = pl.program_id(0)
    nsteps = pl.num_programs(0)

    xb = xs_ref[...]
    logits = jax.lax.dot_general(
        xb, w_ref[...], (((1,), (0,)), ((), ())),
        preferred_element_type=jnp.float32) + b_ref[...]
    masked = logits * mask_ref[...]

    mx = jnp.max(masked, axis=1, keepdims=True)
    ex = jnp.exp(masked - mx)
    lse = jnp.log(jnp.sum(ex, axis=1, keepdims=True)) + mx

    tcol = tgt_ref[0, 0, :][:, None]
    iota = jax.lax.broadcasted_iota(jnp.int32, (ROWS, VOCAB), 1)
    tlog = jnp.sum(jnp.where(iota == tcol, masked, 0.0), axis=1, keepdims=True)

    part = (jnp.sum(lse - tlog) / nrows_total).reshape(1, 1)

    @pl.when(i == 0)
    def _():
        out_ref[...] = jnp.zeros_like(out_ref)

    out_ref[...] += part


def kernel(x, masked_output, W, b, Wv, bv):
    B, L, V = masked_output.shape
    N = B * L
    nsteps = N // ROWS

    xs = x[:, L:2 * L, :].reshape(N, 3)
    tgt = x[:, L + 1:, 0].astype(jnp.int32).reshape(nsteps, 1, ROWS)
    mask2d = masked_output.reshape(N, V)
    b2d = b.reshape(1, V)

    out = pl.pallas_call(
        functools.partial(_ce_body, float(N)),
        grid=(nsteps,),
        in_specs=[
            pl.BlockSpec((ROWS, 3), lambda i: (i, 0)),
            pl.BlockSpec((ROWS, V), lambda i: (i, 0)),
            pl.BlockSpec((1, 1, ROWS), lambda i: (i, 0, 0)),
            pl.BlockSpec((3, V), lambda i: (0, 0)),
            pl.BlockSpec((1, V), lambda i: (0, 0)),
        ],
        out_specs=pl.BlockSpec((1, 1), lambda i: (0, 0)),
        out_shape=jax.ShapeDtypeStruct((1, 1), jnp.float32),
    )(xs, mask2d, tgt, W, b2d)
    return out[0, 0]
